# baseline (speedup 1.0000x reference)
import functools

import jax
import jax.numpy as jnp
from jax import lax
from jax.experimental import pallas as pl
from jax.experimental.pallas import tpu as pltpu
from jax.experimental.pallas import tpu_sc as plsc

D_MODEL = 64
LN_EPS = 1e-5
SEQ = 128
BATCH = 4096
NROWS = SEQ * BATCH

NC = 2
NS = 16
L = 16
NW = NC * NS

ROWS_PER_W = NROWS // NW
CHUNK = 128
NCHUNKS = ROWS_PER_W // CHUNK
NSEG = D_MODEL // L
NBUF = 3
S_PER_W = ROWS_PER_W // BATCH
CHUNKS_PER_S = BATCH // CHUNK


def _lane_sum(v):
    lanes = lax.iota(jnp.int32, L)
    dnums = lax.GatherDimensionNumbers(
        offset_dims=(), collapsed_slice_dims=(0,), start_index_map=(0,)
    )
    for sh in (8, 4, 2, 1):
        perm = lax.gather(
            v, (lanes ^ sh)[:, None], dnums, (1,),
            mode=lax.GatherScatterMode.PROMISE_IN_BOUNDS,
        )
        v = v + perm
    return v


_mesh = plsc.VectorSubcoreMesh(
    core_axis_name="c", subcore_axis_name="s", num_cores=NC, num_subcores=NS
)


@functools.partial(
    pl.kernel,
    out_type=jax.ShapeDtypeStruct((NROWS, D_MODEL), jnp.float32),
    mesh=_mesh,
    scratch_types=[
        pltpu.VMEM((NCHUNKS, CHUNK), jnp.int32),
        pltpu.VMEM((NBUF, CHUNK, D_MODEL), jnp.float32),
        pltpu.VMEM((S_PER_W, D_MODEL), jnp.float32),
        pltpu.VMEM((D_MODEL,), jnp.float32),
        pltpu.VMEM((D_MODEL,), jnp.float32),
        pltpu.SemaphoreType.DMA((NBUF,)),
        pltpu.SemaphoreType.DMA((NBUF,)),
    ],
    compiler_params=pltpu.CompilerParams(use_tc_tiling_on_sc=False),
)
def _sc_embed_ln(x_hbm, tab_hbm, pos_hbm, gamma_hbm, beta_hbm, out_hbm,
                 idx_v, rows_v, pos_v, gamma_v, beta_v, gsem, osem):
    wid = lax.axis_index("s") * NC + lax.axis_index("c")
    base = wid * ROWS_PER_W

    pltpu.sync_copy(x_hbm.at[pl.ds(wid * NCHUNKS, NCHUNKS)], idx_v)
    pltpu.sync_copy(pos_hbm.at[pl.ds(wid * S_PER_W, S_PER_W)], pos_v)
    pltpu.sync_copy(gamma_hbm, gamma_v)
    pltpu.sync_copy(beta_hbm, beta_v)
    gseg = [gamma_v[pl.ds(k * L, L)] for k in range(NSEG)]
    bseg = [beta_v[pl.ds(k * L, L)] for k in range(NSEG)]

    def start_gather(c, t):
        return pltpu.async_copy(tab_hbm.at[idx_v.at[c]], rows_v.at[t],
                                gsem.at[t])

    start_gather(0, 0)
    start_gather(1, 1)

    def process(c, t):
        pltpu.make_async_copy(tab_hbm.at[idx_v.at[c]], rows_v.at[t],
                              gsem.at[t]).wait()
        s_loc = lax.shift_right_logical(c, 5)
        pseg = [pos_v[s_loc, pl.ds(k * L, L)] for k in range(NSEG)]

        def row_body(r, carry2):
            e = [rows_v[t, r, pl.ds(k * L, L)] * 8.0 + pseg[k]
                 for k in range(NSEG)]
            ssum = (e[0] + e[1]) + (e[2] + e[3])
            ssq = ((e[0] * e[0] + e[1] * e[1])
                   + (e[2] * e[2] + e[3] * e[3]))
            mean_v = _lane_sum(ssum) * (1.0 / D_MODEL)
            var_v = _lane_sum(ssq) * (1.0 / D_MODEL) - mean_v * mean_v
            a_v = var_v + LN_EPS
            yi = jnp.full((L,), 0x5F3759DF, jnp.int32) - lax.shift_right_logical(
                lax.bitcast_convert_type(a_v, jnp.int32), 1
            )
            y = lax.bitcast_convert_type(yi, jnp.float32)
            h_v = a_v * -0.5
            y = y * (y * y * h_v + 1.5)
            y = y * (y * y * h_v + 1.5)
            rstd_v = y
            for k in range(NSEG):
                rows_v[t, r, pl.ds(k * L, L)] = (
                    (e[k] - mean_v) * rstd_v * gseg[k] + bseg[k]
                )
            return carry2

        lax.fori_loop(0, CHUNK, row_body, 0, unroll=2)
        return pltpu.async_copy(rows_v.at[t],
                                out_hbm.at[pl.ds(base + c * CHUNK, CHUNK)],
                                osem.at[t])

    def wait_out(c, t):
        pltpu.make_async_copy(rows_v.at[t],
                              out_hbm.at[pl.ds(base + c * CHUNK, CHUNK)],
                              osem.at[t]).wait()

    process(0, 0)
    start_gather(2, 2)

    def chunk_body(c, carry):
        t = c % NBUF
        process(c, t)
        t2 = (c + 2) % NBUF
        wait_out(c - 1, t2)
        start_gather(c + 2, t2)
        return carry

    lax.fori_loop(1, NCHUNKS - 2, chunk_body, 0)

    process(NCHUNKS - 2, (NCHUNKS - 2) % NBUF)
    process(NCHUNKS - 1, (NCHUNKS - 1) % NBUF)
    for c in (NCHUNKS - 3, NCHUNKS - 2, NCHUNKS - 1):
        wait_out(c, c % NBUF)


def kernel(x, token_table, pos_table, ln_gamma, ln_beta):
    x_flat = x.reshape(NW * NCHUNKS, CHUNK).astype(jnp.int32)
    out = _sc_embed_ln(x_flat, token_table, pos_table, ln_gamma, ln_beta)
    return out.reshape(SEQ, BATCH, D_MODEL)


# speedup vs baseline: 1.2720x; 1.2720x over previous
import functools

import jax
import jax.numpy as jnp
from jax import lax
from jax.experimental import pallas as pl
from jax.experimental.pallas import tpu as pltpu
from jax.experimental.pallas import tpu_sc as plsc

D_MODEL = 64
LN_EPS = 1e-5
SEQ = 128
BATCH = 4096
NROWS = SEQ * BATCH

NC = 2
NS = 16
L = 16
NW = NC * NS

ROWS_PER_W = NROWS // NW
CHUNK = 128
NCHUNKS = ROWS_PER_W // CHUNK
NSEG = D_MODEL // L
NBUF = 3
S_PER_W = ROWS_PER_W // BATCH
CHUNKS_PER_S = BATCH // CHUNK


def _lane_sum(v):
    lanes = lax.iota(jnp.int32, L)
    dnums = lax.GatherDimensionNumbers(
        offset_dims=(), collapsed_slice_dims=(0,), start_index_map=(0,)
    )
    for sh in (8, 4, 2, 1):
        perm = lax.gather(
            v, (lanes ^ sh)[:, None], dnums, (1,),
            mode=lax.GatherScatterMode.PROMISE_IN_BOUNDS,
        )
        v = v + perm
    return v


_mesh = plsc.VectorSubcoreMesh(
    core_axis_name="c", subcore_axis_name="s", num_cores=NC, num_subcores=NS
)


@functools.partial(
    pl.kernel,
    out_type=jax.ShapeDtypeStruct((SEQ, BATCH, D_MODEL), jnp.float32),
    mesh=_mesh,
    scratch_types=[
        pltpu.VMEM((S_PER_W, BATCH), jnp.int32),
        pltpu.VMEM((NBUF, CHUNK, D_MODEL), jnp.float32),
        pltpu.VMEM((S_PER_W, D_MODEL), jnp.float32),
        pltpu.VMEM((D_MODEL,), jnp.float32),
        pltpu.VMEM((D_MODEL,), jnp.float32),
        pltpu.SemaphoreType.DMA((NBUF,)),
        pltpu.SemaphoreType.DMA((NBUF,)),
    ],
    compiler_params=pltpu.CompilerParams(use_tc_tiling_on_sc=False),
)
def _sc_embed_ln(x_hbm, tab_hbm, pos_hbm, gamma_hbm, beta_hbm, out_hbm,
                 idx_v, rows_v, pos_v, gamma_v, beta_v, gsem, osem):
    wid = lax.axis_index("s") * NC + lax.axis_index("c")
    s0 = wid * S_PER_W

    pltpu.sync_copy(x_hbm.at[pl.ds(s0, S_PER_W)], idx_v)
    pltpu.sync_copy(pos_hbm.at[pl.ds(s0, S_PER_W)], pos_v)
    pltpu.sync_copy(gamma_hbm, gamma_v)
    pltpu.sync_copy(beta_hbm, beta_v)
    gseg = [gamma_v[pl.ds(k * L, L)] for k in range(NSEG)]
    bseg = [beta_v[pl.ds(k * L, L)] for k in range(NSEG)]

    def sloc(c):
        return lax.shift_right_logical(c, 5)

    def boff(c):
        return (c & (CHUNKS_PER_S - 1)) * CHUNK

    def start_gather(c, t):
        return pltpu.async_copy(
            tab_hbm.at[idx_v.at[sloc(c), pl.ds(boff(c), CHUNK)]],
            rows_v.at[t], gsem.at[t])

    def out_copy_descr(c, t):
        return pltpu.make_async_copy(
            rows_v.at[t],
            out_hbm.at[s0 + sloc(c), pl.ds(boff(c), CHUNK)],
            osem.at[t])

    start_gather(0, 0)
    start_gather(1, 1)

    def process(c, t):
        pltpu.make_async_copy(
            tab_hbm.at[idx_v.at[sloc(c), pl.ds(boff(c), CHUNK)]],
            rows_v.at[t], gsem.at[t]).wait()
        sl = sloc(c)
        pseg = [pos_v[sl, pl.ds(k * L, L)] for k in range(NSEG)]

        @plsc.parallel_loop(0, CHUNK, 1, unroll=4)
        def row_body(r):
            e = [rows_v[t, r, pl.ds(k * L, L)] * 8.0 + pseg[k]
                 for k in range(NSEG)]
            ssum = (e[0] + e[1]) + (e[2] + e[3])
            ssq = ((e[0] * e[0] + e[1] * e[1])
                   + (e[2] * e[2] + e[3] * e[3]))
            mean_v = _lane_sum(ssum) * (1.0 / D_MODEL)
            var_v = _lane_sum(ssq) * (1.0 / D_MODEL) - mean_v * mean_v
            a_v = var_v + LN_EPS
            yi = jnp.full((L,), 0x5F3759DF, jnp.int32) - lax.shift_right_logical(
                lax.bitcast_convert_type(a_v, jnp.int32), 1
            )
            y = lax.bitcast_convert_type(yi, jnp.float32)
            h_v = a_v * -0.5
            y = y * (y * y * h_v + 1.5)
            y = y * (y * y * h_v + 1.5)
            for k in range(NSEG):
                rg = y * gseg[k]
                ob = bseg[k] - mean_v * rg
                rows_v[t, r, pl.ds(k * L, L)] = e[k] * rg + ob
            return None

        out_copy_descr(c, t).start()

    process(0, 0)
    start_gather(2, 2)

    def chunk_body(c, carry):
        t = c % NBUF
        process(c, t)
        t2 = (c + 2) % NBUF
        out_copy_descr(c - 1, t2).wait()
        start_gather(c + 2, t2)
        return carry

    lax.fori_loop(1, NCHUNKS - 2, chunk_body, 0)

    process(NCHUNKS - 2, (NCHUNKS - 2) % NBUF)
    process(NCHUNKS - 1, (NCHUNKS - 1) % NBUF)
    for c in (NCHUNKS - 3, NCHUNKS - 2, NCHUNKS - 1):
        out_copy_descr(c, c % NBUF).wait()


def kernel(x, token_table, pos_table, ln_gamma, ln_beta):
    out = _sc_embed_ln(x.astype(jnp.int32), token_table, pos_table,
                       ln_gamma, ln_beta)
    return out


# speedup vs baseline: 1.2750x; 1.0024x over previous
import functools

import jax
import jax.numpy as jnp
from jax import lax
from jax.experimental import pallas as pl
from jax.experimental.pallas import tpu as pltpu
from jax.experimental.pallas import tpu_sc as plsc

D_MODEL = 64
LN_EPS = 1e-5
SEQ = 128
BATCH = 4096
NROWS = SEQ * BATCH

NC = 2
NS = 16
L = 16
NW = NC * NS

ROWS_PER_W = NROWS // NW
CHUNK = 128
NCHUNKS = ROWS_PER_W // CHUNK
NSEG = D_MODEL // L
NBUF = 3
S_PER_W = ROWS_PER_W // BATCH
CHUNKS_PER_S = BATCH // CHUNK


def _lane_sum(v):
    lanes = lax.iota(jnp.int32, L)
    dnums = lax.GatherDimensionNumbers(
        offset_dims=(), collapsed_slice_dims=(0,), start_index_map=(0,)
    )
    for sh in (8, 4, 2, 1):
        perm = lax.gather(
            v, (lanes ^ sh)[:, None], dnums, (1,),
            mode=lax.GatherScatterMode.PROMISE_IN_BOUNDS,
        )
        v = v + perm
    return v


_mesh = plsc.VectorSubcoreMesh(
    core_axis_name="c", subcore_axis_name="s", num_cores=NC, num_subcores=NS
)


@functools.partial(
    pl.kernel,
    out_type=jax.ShapeDtypeStruct((SEQ, BATCH, D_MODEL), jnp.float32),
    mesh=_mesh,
    scratch_types=[
        pltpu.VMEM((S_PER_W, BATCH), jnp.int32),
        pltpu.VMEM((NBUF, CHUNK, D_MODEL), jnp.float32),
        pltpu.VMEM((S_PER_W, D_MODEL), jnp.float32),
        pltpu.VMEM((D_MODEL,), jnp.float32),
        pltpu.VMEM((D_MODEL,), jnp.float32),
        pltpu.SemaphoreType.DMA((NBUF,)),
        pltpu.SemaphoreType.DMA((NBUF,)),
    ],
    compiler_params=pltpu.CompilerParams(
        use_tc_tiling_on_sc=False, skip_device_barrier=True
    ),
)
def _sc_embed_ln(x_hbm, tab_hbm, pos_hbm, gamma_hbm, beta_hbm, out_hbm,
                 idx_v, rows_v, pos_v, gamma_v, beta_v, gsem, osem):
    wid = lax.axis_index("s") * NC + lax.axis_index("c")
    s0 = wid * S_PER_W

    pltpu.sync_copy(x_hbm.at[pl.ds(s0, S_PER_W)], idx_v)
    pltpu.sync_copy(pos_hbm.at[pl.ds(s0, S_PER_W)], pos_v)
    pltpu.sync_copy(gamma_hbm, gamma_v)
    pltpu.sync_copy(beta_hbm, beta_v)
    gseg = [gamma_v[pl.ds(k * L, L)] for k in range(NSEG)]
    bseg = [beta_v[pl.ds(k * L, L)] for k in range(NSEG)]

    def sloc(c):
        return lax.shift_right_logical(c, 5)

    def boff(c):
        return (c & (CHUNKS_PER_S - 1)) * CHUNK

    def start_gather(c, t):
        return pltpu.async_copy(
            tab_hbm.at[idx_v.at[sloc(c), pl.ds(boff(c), CHUNK)]],
            rows_v.at[t], gsem.at[t])

    def out_copy_descr(c, t):
        return pltpu.make_async_copy(
            rows_v.at[t],
            out_hbm.at[s0 + sloc(c), pl.ds(boff(c), CHUNK)],
            osem.at[t])

    start_gather(0, 0)
    start_gather(1, 1)

    def process(c, t):
        pltpu.make_async_copy(
            tab_hbm.at[idx_v.at[sloc(c), pl.ds(boff(c), CHUNK)]],
            rows_v.at[t], gsem.at[t]).wait()
        sl = sloc(c)
        pseg = [pos_v[sl, pl.ds(k * L, L)] for k in range(NSEG)]

        @plsc.parallel_loop(0, CHUNK, 1, unroll=4)
        def row_body(r):
            e = [rows_v[t, r, pl.ds(k * L, L)] * 8.0 + pseg[k]
                 for k in range(NSEG)]
            ssum = (e[0] + e[1]) + (e[2] + e[3])
            ssq = ((e[0] * e[0] + e[1] * e[1])
                   + (e[2] * e[2] + e[3] * e[3]))
            mean_v = _lane_sum(ssum) * (1.0 / D_MODEL)
            var_v = _lane_sum(ssq) * (1.0 / D_MODEL) - mean_v * mean_v
            a_v = var_v + LN_EPS
            yi = jnp.full((L,), 0x5F3759DF, jnp.int32) - lax.shift_right_logical(
                lax.bitcast_convert_type(a_v, jnp.int32), 1
            )
            y = lax.bitcast_convert_type(yi, jnp.float32)
            h_v = a_v * -0.5
            y = y * (y * y * h_v + 1.5)
            y = y * (y * y * h_v + 1.5)
            for k in range(NSEG):
                rg = y * gseg[k]
                ob = bseg[k] - mean_v * rg
                rows_v[t, r, pl.ds(k * L, L)] = e[k] * rg + ob
            return None

        out_copy_descr(c, t).start()

    process(0, 0)
    start_gather(2, 2)

    def chunk_body(c, carry):
        t = c % NBUF
        process(c, t)
        t2 = (c + 2) % NBUF
        out_copy_descr(c - 1, t2).wait()
        start_gather(c + 2, t2)
        return carry

    lax.fori_loop(1, NCHUNKS - 2, chunk_body, 0)

    process(NCHUNKS - 2, (NCHUNKS - 2) % NBUF)
    process(NCHUNKS - 1, (NCHUNKS - 1) % NBUF)
    for c in (NCHUNKS - 3, NCHUNKS - 2, NCHUNKS - 1):
        out_copy_descr(c, c % NBUF).wait()


def kernel(x, token_table, pos_table, ln_gamma, ln_beta):
    out = _sc_embed_ln(x.astype(jnp.int32), token_table, pos_table,
                       ln_gamma, ln_beta)
    return out
